# baseline (speedup 1.0000x reference)
import functools

import jax
import jax.numpy as jnp
from jax import lax
from jax.experimental import pallas as pl
from jax.experimental.pallas import tpu as pltpu
from jax.experimental.pallas import tpu_sc as plsc

N = 10000
E = 320000
D = 128
HD = 64
HID = 32
EPS = 1e-5

NC = 2
NS = 16
NW = NC * NS

EP = 327680
CH = 80
NSUB = 64
NCHUNK = EP // CH
NSUP = NCHUNK // NSUB
VROW = NSUB * CH // 128
NVR = EP // 128
NP = 10240

_MESH = plsc.VectorSubcoreMesh(core_axis_name="c", subcore_axis_name="s")


def _diff_body(x_hbm, rows_hbm, cols_hbm, out_hbm,
               rbuf, cbuf, bra, bca, brb, bcb, sra, sca, srb, scb):
    cid = lax.axis_index("c")
    sid = lax.axis_index("s")
    wid = sid * NC + cid

    def _absdiff(br, bc):
        def edge(e, _):
            for k in range(D // 16):
                sl = pl.ds(k * 16, 16)
                br[e, sl] = jnp.abs(br[e, sl] - bc[e, sl])
            return 0

        lax.fori_loop(0, CH, edge, 0)

    def sup(s, _):
        t = wid + NW * s

        @pl.when(t < NSUP)
        def _():
            crow = t * NSUB
            pltpu.sync_copy(rows_hbm.at[pl.ds(crow, NSUB)], rbuf)
            pltpu.sync_copy(cols_hbm.at[pl.ds(crow, NSUB)], cbuf)
            pltpu.async_copy(x_hbm.at[rbuf.at[0]], bra, sra)
            pltpu.async_copy(x_hbm.at[cbuf.at[0]], bca, sca)

            def pair(jj, _):
                j0 = 2 * jj
                j1 = j0 + 1
                pltpu.async_copy(x_hbm.at[rbuf.at[j1]], brb, srb)
                pltpu.async_copy(x_hbm.at[cbuf.at[j1]], bcb, scb)
                pltpu.make_async_copy(x_hbm.at[rbuf.at[j0]], bra, sra).wait()
                pltpu.make_async_copy(x_hbm.at[cbuf.at[j0]], bca, sca).wait()
                _absdiff(bra, bca)
                pltpu.sync_copy(bra, out_hbm.at[pl.ds((crow + j0) * CH, CH)])

                @pl.when(j0 + 2 < NSUB)
                def _():
                    pltpu.async_copy(x_hbm.at[rbuf.at[j0 + 2]], bra, sra)
                    pltpu.async_copy(x_hbm.at[cbuf.at[j0 + 2]], bca, sca)

                pltpu.make_async_copy(x_hbm.at[rbuf.at[j1]], brb, srb).wait()
                pltpu.make_async_copy(x_hbm.at[cbuf.at[j1]], bcb, scb).wait()
                _absdiff(brb, bcb)
                pltpu.sync_copy(brb, out_hbm.at[pl.ds((crow + j1) * CH, CH)])
                return 0

            lax.fori_loop(0, NSUB // 2, pair, 0)
        return 0

    lax.fori_loop(0, -(-NSUP // NW), sup, 0)


_diff_call = functools.partial(
    pl.kernel,
    out_type=jax.ShapeDtypeStruct((EP, D), jnp.float32),
    mesh=_MESH,
    compiler_params=pltpu.CompilerParams(use_tc_tiling_on_sc=False),
    scratch_types=[
        pltpu.VMEM((NSUB, CH), jnp.int32),
        pltpu.VMEM((NSUB, CH), jnp.int32),
        pltpu.VMEM((CH, D), jnp.float32),
        pltpu.VMEM((CH, D), jnp.float32),
        pltpu.VMEM((CH, D), jnp.float32),
        pltpu.VMEM((CH, D), jnp.float32),
        pltpu.SemaphoreType.DMA,
        pltpu.SemaphoreType.DMA,
        pltpu.SemaphoreType.DMA,
        pltpu.SemaphoreType.DMA,
    ],
)(_diff_body)


MLP_BLK = 4096
MROW = MLP_BLK // 128


def _mlp_body(diff_ref, w1_ref, b1_ref, w2_ref, b2_ref, o1_ref, o2_ref):
    d = diff_ref[...]
    h = jnp.dot(d, w1_ref[...], preferred_element_type=jnp.float32) + b1_ref[...]
    h = jnp.maximum(h, 0.0)
    u = jnp.dot(h, w2_ref[...], preferred_element_type=jnp.float32) + b2_ref[...]
    sig = 1.0 / (1.0 + jnp.exp(-u))
    o1_ref[...] = sig[:, 0].reshape(MROW, 128)
    o2_ref[...] = sig[:, 1].reshape(MROW, 128)


def _mlp_call(diff, w1cat, b1cat, w2blk, b2cat):
    grid = (EP // MLP_BLK,)
    return pl.pallas_call(
        _mlp_body,
        grid=grid,
        in_specs=[
            pl.BlockSpec((MLP_BLK, D), lambda i: (i, 0)),
            pl.BlockSpec((D, 2 * HID), lambda i: (0, 0)),
            pl.BlockSpec((1, 2 * HID), lambda i: (0, 0)),
            pl.BlockSpec((2 * HID, 2), lambda i: (0, 0)),
            pl.BlockSpec((1, 2), lambda i: (0, 0)),
        ],
        out_specs=[pl.BlockSpec((MROW, 128), lambda i: (i, 0)),
                   pl.BlockSpec((MROW, 128), lambda i: (i, 0))],
        out_shape=[jax.ShapeDtypeStruct((NVR, 128), jnp.float32),
                   jax.ShapeDtypeStruct((NVR, 128), jnp.float32)],
    )(diff, w1cat, b1cat, w2blk, b2cat)


def _spmm_body(fL, fR, rows_hbm, cols_hbm, v1_hbm, v2_hbm,
               o1L, o1R, o2L, o2R,
               acc1, acc2, rbuf, cbuf, v1b, v2b, ga, gb, s1, s2, sema, semb):
    cid = lax.axis_index("c")
    sid = lax.axis_index("s")
    zr = NP // NS

    def zrow(e, _):
        for k in range(HD // 16):
            s1[e, pl.ds(k * 16, 16)] = jnp.zeros((16,), jnp.float32)
        return 0

    lax.fori_loop(0, CH, zrow, 0)

    def zcp(i, _):
        zsl = pl.ds(sid * zr + i * CH, CH)
        pltpu.sync_copy(s1, acc1.at[zsl])
        pltpu.sync_copy(s1, acc2.at[zsl])
        return 0

    lax.fori_loop(0, zr // CH, zcp, 0)
    plsc.subcore_barrier()

    def sup(s, _):
        t = sid + NS * s

        @pl.when(t < NSUP)
        def _():
            _spmm_super(t, cid, fL, fR, rows_hbm, cols_hbm, v1_hbm, v2_hbm,
                        acc1, acc2, rbuf, cbuf, v1b, v2b, ga, gb, s1, s2,
                        sema, semb)
        return 0

    lax.fori_loop(0, -(-NSUP // NS), sup, 0)
    plsc.subcore_barrier()

    osl = pl.ds(sid * zr, zr)

    @pl.when(cid == 0)
    def _():
        pltpu.sync_copy(acc1.at[osl], o1L.at[osl])
        pltpu.sync_copy(acc2.at[osl], o2L.at[osl])

    @pl.when(cid == 1)
    def _():
        pltpu.sync_copy(acc1.at[osl], o1R.at[osl])
        pltpu.sync_copy(acc2.at[osl], o2R.at[osl])


def _spmm_super(t, cid, fL, fR, rows_hbm, cols_hbm, v1_hbm, v2_hbm,
                acc1, acc2, rbuf, cbuf, v1b, v2b, ga, gb, s1, s2, sema, semb):
    crow = t * NSUB
    pltpu.sync_copy(rows_hbm.at[pl.ds(crow, NSUB)], rbuf)
    pltpu.sync_copy(cols_hbm.at[pl.ds(crow, NSUB)], cbuf)
    pltpu.sync_copy(v1_hbm.at[pl.ds(t * VROW, VROW)], v1b)
    pltpu.sync_copy(v2_hbm.at[pl.ds(t * VROW, VROW)], v2b)

    def gissue(j, buf, sem):
        @pl.when(cid == 0)
        def _():
            pltpu.async_copy(fL.at[cbuf.at[j]], buf, sem)

        @pl.when(cid == 1)
        def _():
            pltpu.async_copy(fR.at[cbuf.at[j]], buf, sem)

    def gwait(j, buf, sem):
        @pl.when(cid == 0)
        def _():
            pltpu.make_async_copy(fL.at[cbuf.at[j]], buf, sem).wait()

        @pl.when(cid == 1)
        def _():
            pltpu.make_async_copy(fR.at[cbuf.at[j]], buf, sem).wait()

    def scale_scatter(j, g):
        def eblk(eb, _):
            m = 5 * j + eb
            vr = m // 8
            vo = 16 * (m - 8 * vr)
            v1v = v1b[vr, pl.ds(vo, 16)]
            v2v = v2b[vr, pl.ds(vo, 16)]
            for l in range(16):
                e = eb * 16 + l
                a1 = v1v[l]
                a2 = v2v[l]
                for k in range(HD // 16):
                    sl = pl.ds(k * 16, 16)
                    r = g[e, sl]
                    s1[e, sl] = r * a1
                    s2[e, sl] = r * a2
            return 0

        lax.fori_loop(0, CH // 16, eblk, 0)
        pltpu.sync_copy(s1, acc1.at[rbuf.at[j]], add=True)
        pltpu.sync_copy(s2, acc2.at[rbuf.at[j]], add=True)

    gissue(0, ga, sema)

    def pair(jj, _):
        j0 = 2 * jj
        j1 = j0 + 1
        gissue(j1, gb, semb)
        gwait(j0, ga, sema)
        scale_scatter(j0, ga)

        @pl.when(j0 + 2 < NSUB)
        def _():
            gissue(j0 + 2, ga, sema)

        gwait(j1, gb, semb)
        scale_scatter(j1, gb)
        return 0

    lax.fori_loop(0, NSUB // 2, pair, 0)


_spmm_call = functools.partial(
    pl.kernel,
    out_type=[jax.ShapeDtypeStruct((NP, HD), jnp.float32)] * 4,
    mesh=_MESH,
    compiler_params=pltpu.CompilerParams(use_tc_tiling_on_sc=False),
    scratch_types=[
        pltpu.VMEM_SHARED((NP, HD), jnp.float32),
        pltpu.VMEM_SHARED((NP, HD), jnp.float32),
        pltpu.VMEM((NSUB, CH), jnp.int32),
        pltpu.VMEM((NSUB, CH), jnp.int32),
        pltpu.VMEM((VROW, 128), jnp.float32),
        pltpu.VMEM((VROW, 128), jnp.float32),
        pltpu.VMEM((CH, HD), jnp.float32),
        pltpu.VMEM((CH, HD), jnp.float32),
        pltpu.VMEM((CH, HD), jnp.float32),
        pltpu.VMEM((CH, HD), jnp.float32),
        pltpu.SemaphoreType.DMA,
        pltpu.SemaphoreType.DMA,
    ],
)(_spmm_body)


def _conv_body(x_ref, a_ref, b_ref, c_ref, d_ref,
               wx, wa, wb, wc, wd, bias, gamma, beta, res_ref, out_ref):
    u = jnp.dot(x_ref[...], wx[...], preferred_element_type=jnp.float32)
    u += jnp.dot(a_ref[...], wa[...], preferred_element_type=jnp.float32)
    u += jnp.dot(b_ref[...], wb[...], preferred_element_type=jnp.float32)
    u += jnp.dot(c_ref[...], wc[...], preferred_element_type=jnp.float32)
    u += jnp.dot(d_ref[...], wd[...], preferred_element_type=jnp.float32)
    u += bias[...]
    mean = jnp.mean(u, axis=0, keepdims=True)
    var = jnp.mean((u - mean) * (u - mean), axis=0, keepdims=True)
    h = gamma[...] * (u - mean) * lax.rsqrt(var + EPS) + beta[...]
    h += res_ref[...]
    out_ref[...] = jnp.maximum(h, 0.0)


def _conv_call(xin, a, b, c, d, fc_w, fc_b, gamma, beta, res):
    wx = fc_w[0:D]
    wa = fc_w[D:D + HD]
    wb = fc_w[D + HD:2 * D]
    wc = fc_w[2 * D:2 * D + HD]
    wd = fc_w[2 * D + HD:3 * D]
    return pl.pallas_call(
        _conv_body,
        out_shape=jax.ShapeDtypeStruct((N, D), jnp.float32),
    )(xin, a, b, c, d, wx, wa, wb, wc, wd,
      fc_b[None], gamma[None], beta[None], res)


def kernel(x, edge_index, edge_values,
           wc1_w1, wc1_b1, wc1_w2, wc1_b2,
           wc2_w1, wc2_b1, wc2_w2, wc2_b2,
           conv1_fc_w, conv1_fc_b, conv1_gamma, conv1_beta,
           conv2_fc_w, conv2_fc_b, conv2_gamma, conv2_beta):
    pad = EP - E
    rows_d = jnp.concatenate(
        [edge_index[0], jnp.zeros((pad,), jnp.int32)]).reshape(NCHUNK, CH)
    cols_p = jnp.concatenate(
        [edge_index[1], jnp.zeros((pad,), jnp.int32)]).reshape(NCHUNK, CH)
    rows_s = jnp.concatenate(
        [edge_index[0], jnp.full((pad,), NP - 1, jnp.int32)]).reshape(NCHUNK, CH)

    diff = _diff_call(x, rows_d, cols_p)

    w1cat = jnp.concatenate([wc1_w1, wc2_w1], axis=1)
    b1cat = jnp.concatenate([wc1_b1, wc2_b1])[None]
    w2blk = jnp.zeros((2 * HID, 2), jnp.float32)
    w2blk = w2blk.at[:HID, 0].set(wc1_w2[:, 0]).at[HID:, 1].set(wc2_w2[:, 0])
    b2cat = jnp.stack([wc1_b2[0], wc2_b2[0]])[None]
    v1, v2 = _mlp_call(diff, w1cat, b1cat, w2blk, b2cat)

    xL = x[:, :HD]
    xR = x[:, HD:]
    y1L, y1R, y2L, y2R = (
        o[:N] for o in _spmm_call(xL, xR, rows_s, cols_p, v1, v2))

    zero_res = jnp.zeros((N, D), jnp.float32)
    h = _conv_call(x, y1L, y1R, y2L, y2R,
                   conv1_fc_w, conv1_fc_b, conv1_gamma, conv1_beta, zero_res)

    hL = h[:, :HD]
    hR = h[:, HD:]
    z1L, z1R, z2L, z2R = (
        o[:N] for o in _spmm_call(hL, hR, rows_s, cols_p, v1, v2))

    out = _conv_call(h, z1L, z1R, z2L, z2R,
                     conv2_fc_w, conv2_fc_b, conv2_gamma, conv2_beta, x)
    return out


# speedup vs baseline: 1.9837x; 1.9837x over previous
import functools

import jax
import jax.numpy as jnp
from jax import lax
from jax.experimental import pallas as pl
from jax.experimental.pallas import tpu as pltpu
from jax.experimental.pallas import tpu_sc as plsc

N = 10000
E = 320000
D = 128
HD = 64
HID = 32
EPS = 1e-5

NC = 2
NS = 16
NW = NC * NS

EP = 327680
CH = 80
NSUB = 64
NCHUNK = EP // CH
NSUP = NCHUNK // NSUB
VROW = NSUB * CH // 128
NVR = EP // 128
NP = 10240

_MESH = plsc.VectorSubcoreMesh(core_axis_name="c", subcore_axis_name="s")


def _diff_body(x_hbm, rows_hbm, cols_hbm, out_hbm,
               rbuf, cbuf, bra, bca, brb, bcb, sra, sca, srb, scb):
    cid = lax.axis_index("c")
    sid = lax.axis_index("s")
    wid = sid * NC + cid

    def _absdiff(br, bc):
        def edge(e, _):
            for k in range(D // 16):
                sl = pl.ds(k * 16, 16)
                br[e, sl] = jnp.abs(br[e, sl] - bc[e, sl])
            return 0

        lax.fori_loop(0, CH, edge, 0)

    def sup(s, _):
        t = wid + NW * s

        @pl.when(t < NSUP)
        def _():
            crow = t * NSUB
            pltpu.sync_copy(rows_hbm.at[pl.ds(crow, NSUB)], rbuf)
            pltpu.sync_copy(cols_hbm.at[pl.ds(crow, NSUB)], cbuf)
            pltpu.async_copy(x_hbm.at[rbuf.at[0]], bra, sra)
            pltpu.async_copy(x_hbm.at[cbuf.at[0]], bca, sca)

            def pair(jj, _):
                j0 = 2 * jj
                j1 = j0 + 1
                pltpu.async_copy(x_hbm.at[rbuf.at[j1]], brb, srb)
                pltpu.async_copy(x_hbm.at[cbuf.at[j1]], bcb, scb)
                pltpu.make_async_copy(x_hbm.at[rbuf.at[j0]], bra, sra).wait()
                pltpu.make_async_copy(x_hbm.at[cbuf.at[j0]], bca, sca).wait()
                _absdiff(bra, bca)
                pltpu.sync_copy(bra, out_hbm.at[pl.ds((crow + j0) * CH, CH)])

                @pl.when(j0 + 2 < NSUB)
                def _():
                    pltpu.async_copy(x_hbm.at[rbuf.at[j0 + 2]], bra, sra)
                    pltpu.async_copy(x_hbm.at[cbuf.at[j0 + 2]], bca, sca)

                pltpu.make_async_copy(x_hbm.at[rbuf.at[j1]], brb, srb).wait()
                pltpu.make_async_copy(x_hbm.at[cbuf.at[j1]], bcb, scb).wait()
                _absdiff(brb, bcb)
                pltpu.sync_copy(brb, out_hbm.at[pl.ds((crow + j1) * CH, CH)])
                return 0

            lax.fori_loop(0, NSUB // 2, pair, 0)
        return 0

    lax.fori_loop(0, -(-NSUP // NW), sup, 0)


_diff_call = functools.partial(
    pl.kernel,
    out_type=jax.ShapeDtypeStruct((EP, D), jnp.float32),
    mesh=_MESH,
    compiler_params=pltpu.CompilerParams(use_tc_tiling_on_sc=False),
    scratch_types=[
        pltpu.VMEM((NSUB, CH), jnp.int32),
        pltpu.VMEM((NSUB, CH), jnp.int32),
        pltpu.VMEM((CH, D), jnp.float32),
        pltpu.VMEM((CH, D), jnp.float32),
        pltpu.VMEM((CH, D), jnp.float32),
        pltpu.VMEM((CH, D), jnp.float32),
        pltpu.SemaphoreType.DMA,
        pltpu.SemaphoreType.DMA,
        pltpu.SemaphoreType.DMA,
        pltpu.SemaphoreType.DMA,
    ],
)(_diff_body)


MLP_BLK = 4096
MROW = MLP_BLK // 128


def _mlp_body(diff_ref, w1_ref, b1_ref, w2_ref, b2_ref, o1_ref, o2_ref):
    d = diff_ref[...]
    h = jnp.dot(d, w1_ref[...], preferred_element_type=jnp.float32) + b1_ref[...]
    h = jnp.maximum(h, 0.0)
    u = jnp.dot(h, w2_ref[...], preferred_element_type=jnp.float32) + b2_ref[...]
    sig = 1.0 / (1.0 + jnp.exp(-u))
    o1_ref[...] = sig[:, 0].reshape(MROW, 128)
    o2_ref[...] = sig[:, 1].reshape(MROW, 128)


def _mlp_call(diff, w1cat, b1cat, w2blk, b2cat):
    grid = (EP // MLP_BLK,)
    return pl.pallas_call(
        _mlp_body,
        grid=grid,
        in_specs=[
            pl.BlockSpec((MLP_BLK, D), lambda i: (i, 0)),
            pl.BlockSpec((D, 2 * HID), lambda i: (0, 0)),
            pl.BlockSpec((1, 2 * HID), lambda i: (0, 0)),
            pl.BlockSpec((2 * HID, 2), lambda i: (0, 0)),
            pl.BlockSpec((1, 2), lambda i: (0, 0)),
        ],
        out_specs=[pl.BlockSpec((MROW, 128), lambda i: (i, 0)),
                   pl.BlockSpec((MROW, 128), lambda i: (i, 0))],
        out_shape=[jax.ShapeDtypeStruct((NVR, 128), jnp.float32),
                   jax.ShapeDtypeStruct((NVR, 128), jnp.float32)],
    )(diff, w1cat, b1cat, w2blk, b2cat)


def _spmm_body(fL, fR, rows_hbm, cols_hbm, v1_hbm, v2_hbm,
               o1L, o1R, o2L, o2R,
               acc1, acc2, rbuf, cbuf, v1b, v2b, ga, gb, s1, s2, sema, semb):
    cid = lax.axis_index("c")
    sid = lax.axis_index("s")
    zr = NP // NS

    def zrow(e, _):
        for k in range(HD // 16):
            s1[e, pl.ds(k * 16, 16)] = jnp.zeros((16,), jnp.float32)
        return 0

    lax.fori_loop(0, CH, zrow, 0)

    def zcp(i, _):
        zsl = pl.ds(sid * zr + i * CH, CH)
        pltpu.sync_copy(s1, acc1.at[zsl])
        pltpu.sync_copy(s1, acc2.at[zsl])
        return 0

    lax.fori_loop(0, zr // CH, zcp, 0)
    plsc.subcore_barrier()

    def sup(s, _):
        t = sid + NS * s

        @pl.when(t < NSUP)
        def _():
            _spmm_super(t, cid, fL, fR, rows_hbm, cols_hbm, v1_hbm, v2_hbm,
                        acc1, acc2, rbuf, cbuf, v1b, v2b, ga, gb, s1, s2,
                        sema, semb)
        return 0

    lax.fori_loop(0, -(-NSUP // NS), sup, 0)
    plsc.subcore_barrier()

    osl = pl.ds(sid * zr, zr)

    @pl.when(cid == 0)
    def _():
        pltpu.sync_copy(acc1.at[osl], o1L.at[osl])
        pltpu.sync_copy(acc2.at[osl], o2L.at[osl])

    @pl.when(cid == 1)
    def _():
        pltpu.sync_copy(acc1.at[osl], o1R.at[osl])
        pltpu.sync_copy(acc2.at[osl], o2R.at[osl])


def _spmm_super(t, cid, fL, fR, rows_hbm, cols_hbm, v1_hbm, v2_hbm,
                acc1, acc2, rbuf, cbuf, v1b, v2b, ga, gb, s1, s2, sema, semb):
    crow = t * NSUB
    pltpu.sync_copy(rows_hbm.at[pl.ds(crow, NSUB)], rbuf)
    pltpu.sync_copy(cols_hbm.at[pl.ds(crow, NSUB)], cbuf)
    pltpu.sync_copy(v1_hbm.at[pl.ds(t * VROW, VROW)], v1b)
    pltpu.sync_copy(v2_hbm.at[pl.ds(t * VROW, VROW)], v2b)

    def gissue(j, buf, sem):
        @pl.when(cid == 0)
        def _():
            pltpu.async_copy(fL.at[cbuf.at[j]], buf, sem)

        @pl.when(cid == 1)
        def _():
            pltpu.async_copy(fR.at[cbuf.at[j]], buf, sem)

    def gwait(j, buf, sem):
        @pl.when(cid == 0)
        def _():
            pltpu.make_async_copy(fL.at[cbuf.at[j]], buf, sem).wait()

        @pl.when(cid == 1)
        def _():
            pltpu.make_async_copy(fR.at[cbuf.at[j]], buf, sem).wait()

    def scale_scatter(j, g):
        def eblk(eb, _):
            m = 5 * j + eb
            vr = m // 8
            vo = 16 * (m - 8 * vr)
            v1v = v1b[vr, pl.ds(vo, 16)]
            v2v = v2b[vr, pl.ds(vo, 16)]
            for l in range(16):
                e = eb * 16 + l
                a1 = v1v[l]
                a2 = v2v[l]
                for k in range(HD // 16):
                    sl = pl.ds(k * 16, 16)
                    r = g[e, sl]
                    s1[e, sl] = r * a1
                    s2[e, sl] = r * a2
            return 0

        lax.fori_loop(0, CH // 16, eblk, 0)
        pltpu.sync_copy(s1, acc1.at[rbuf.at[j]], add=True)
        pltpu.sync_copy(s2, acc2.at[rbuf.at[j]], add=True)

    gissue(0, ga, sema)

    def pair(jj, _):
        j0 = 2 * jj
        j1 = j0 + 1
        gissue(j1, gb, semb)
        gwait(j0, ga, sema)
        scale_scatter(j0, ga)

        @pl.when(j0 + 2 < NSUB)
        def _():
            gissue(j0 + 2, ga, sema)

        gwait(j1, gb, semb)
        scale_scatter(j1, gb)
        return 0

    lax.fori_loop(0, NSUB // 2, pair, 0)


_spmm_call = functools.partial(
    pl.kernel,
    out_type=[jax.ShapeDtypeStruct((NP, HD), jnp.float32)] * 4,
    mesh=_MESH,
    compiler_params=pltpu.CompilerParams(use_tc_tiling_on_sc=False),
    scratch_types=[
        pltpu.VMEM_SHARED((NP, HD), jnp.float32),
        pltpu.VMEM_SHARED((NP, HD), jnp.float32),
        pltpu.VMEM((NSUB, CH), jnp.int32),
        pltpu.VMEM((NSUB, CH), jnp.int32),
        pltpu.VMEM((VROW, 128), jnp.float32),
        pltpu.VMEM((VROW, 128), jnp.float32),
        pltpu.VMEM((CH, HD), jnp.float32),
        pltpu.VMEM((CH, HD), jnp.float32),
        pltpu.VMEM((CH, HD), jnp.float32),
        pltpu.VMEM((CH, HD), jnp.float32),
        pltpu.SemaphoreType.DMA,
        pltpu.SemaphoreType.DMA,
    ],
)(_spmm_body)


def _conv_body(x_ref, a_ref, b_ref, c_ref, d_ref,
               wx, wa, wb, wc, wd, bias, gamma, beta, res_ref, out_ref):
    u = jnp.dot(x_ref[...], wx[...], preferred_element_type=jnp.float32)
    u += jnp.dot(a_ref[...], wa[...], preferred_element_type=jnp.float32)
    u += jnp.dot(b_ref[...], wb[...], preferred_element_type=jnp.float32)
    u += jnp.dot(c_ref[...], wc[...], preferred_element_type=jnp.float32)
    u += jnp.dot(d_ref[...], wd[...], preferred_element_type=jnp.float32)
    u += bias[...]
    mean = jnp.mean(u, axis=0, keepdims=True)
    var = jnp.mean((u - mean) * (u - mean), axis=0, keepdims=True)
    h = gamma[...] * (u - mean) * lax.rsqrt(var + EPS) + beta[...]
    h += res_ref[...]
    out_ref[...] = jnp.maximum(h, 0.0)


def _conv_call(xin, a, b, c, d, fc_w, fc_b, gamma, beta, res):
    wx = fc_w[0:D]
    wa = fc_w[D:D + HD]
    wb = fc_w[D + HD:2 * D]
    wc = fc_w[2 * D:2 * D + HD]
    wd = fc_w[2 * D + HD:3 * D]
    return pl.pallas_call(
        _conv_body,
        out_shape=jax.ShapeDtypeStruct((N, D), jnp.float32),
    )(xin, a, b, c, d, wx, wa, wb, wc, wd,
      fc_b[None], gamma[None], beta[None], res)


def kernel(x, edge_index, edge_values,
           wc1_w1, wc1_b1, wc1_w2, wc1_b2,
           wc2_w1, wc2_b1, wc2_w2, wc2_b2,
           conv1_fc_w, conv1_fc_b, conv1_gamma, conv1_beta,
           conv2_fc_w, conv2_fc_b, conv2_gamma, conv2_beta):
    pad = EP - E
    spread = jnp.arange(pad, dtype=jnp.int32)
    rows_d = jnp.concatenate(
        [edge_index[0], spread % N]).reshape(NCHUNK, CH)
    cols_p = jnp.concatenate(
        [edge_index[1], spread % N]).reshape(NCHUNK, CH)
    rows_s = jnp.concatenate(
        [edge_index[0], N + spread % (NP - N)]).reshape(NCHUNK, CH)

    diff = _diff_call(x, rows_d, cols_p)

    w1cat = jnp.concatenate([wc1_w1, wc2_w1], axis=1)
    b1cat = jnp.concatenate([wc1_b1, wc2_b1])[None]
    w2blk = jnp.zeros((2 * HID, 2), jnp.float32)
    w2blk = w2blk.at[:HID, 0].set(wc1_w2[:, 0]).at[HID:, 1].set(wc2_w2[:, 0])
    b2cat = jnp.stack([wc1_b2[0], wc2_b2[0]])[None]
    v1, v2 = _mlp_call(diff, w1cat, b1cat, w2blk, b2cat)

    xL = x[:, :HD]
    xR = x[:, HD:]
    y1L, y1R, y2L, y2R = (
        o[:N] for o in _spmm_call(xL, xR, rows_s, cols_p, v1, v2))

    zero_res = jnp.zeros((N, D), jnp.float32)
    h = _conv_call(x, y1L, y1R, y2L, y2R,
                   conv1_fc_w, conv1_fc_b, conv1_gamma, conv1_beta, zero_res)

    hL = h[:, :HD]
    hR = h[:, HD:]
    z1L, z1R, z2L, z2R = (
        o[:N] for o in _spmm_call(hL, hR, rows_s, cols_p, v1, v2))

    out = _conv_call(h, z1L, z1R, z2L, z2R,
                     conv2_fc_w, conv2_fc_b, conv2_gamma, conv2_beta, x)
    return out
